# Initial kernel scaffold; baseline (speedup 1.0000x reference)
import jax
import jax.numpy as jnp
from jax.experimental import pallas as pl


def kernel(inputs, W):
    raise NotImplementedError("write your pallas kernel here")


# speedup vs baseline: 4.0808x; 4.0808x over previous
import jax
import jax.numpy as jnp
from jax.experimental import pallas as pl
from jax.experimental.pallas import tpu as pltpu

_NUM_EMBEDDINGS = 1024
_EMBEDDING_DIM = 256
_COMMITMENT_COST = 0.25
_BLOCK_M = 512


def _vq_kernel(x_ref, w_ref, q_ref, enc_ref, idx_ref, dist_ref,
               loss_ref, ppl_ref, cnt_acc, ls_acc):
    i = pl.program_id(0)
    nblk = pl.num_programs(0)
    x = x_ref[...]
    w = w_ref[...]

    x2 = jnp.sum(x * x, axis=1, keepdims=True)
    w2 = jnp.sum(w * w, axis=1)
    xwt = jax.lax.dot_general(
        x, w, (((1,), (1,)), ((), ())),
        preferred_element_type=jnp.float32)
    dist = x2 + w2[None, :] - 2.0 * xwt
    dist_ref[...] = dist

    k_iota = jax.lax.broadcasted_iota(jnp.int32, dist.shape, 1)
    minv = jnp.min(dist, axis=1, keepdims=True)
    idx = jnp.min(jnp.where(dist == minv, k_iota, dist.shape[1]),
                  axis=1).astype(jnp.int32)
    idx_ref[...] = idx[:, None]

    onehot = (k_iota == idx[:, None]).astype(jnp.float32)
    enc_ref[...] = onehot

    q = jax.lax.dot_general(
        onehot, w, (((1,), (0,)), ((), ())),
        preferred_element_type=jnp.float32)
    q_ref[...] = x + (q - x)

    @pl.when(i == 0)
    def _():
        cnt_acc[...] = jnp.zeros_like(cnt_acc)
        ls_acc[...] = jnp.zeros_like(ls_acc)

    diff = q - x
    cnt_acc[...] += jnp.sum(onehot, axis=0, keepdims=True)
    ls_acc[...] += jnp.sum(diff * diff)[None, None]

    @pl.when(i == nblk - 1)
    def _():
        n_tok = nblk * x.shape[0]
        total = n_tok * x.shape[1]
        loss_ref[...] = _COMMITMENT_COST / total * ls_acc[...]
        avg = cnt_acc[...] / n_tok
        ent = jnp.sum(avg * jnp.log(avg + 1e-10), axis=1, keepdims=True)
        ppl_ref[...] = jnp.exp(-ent)


def kernel(inputs, W):
    input_shape = inputs.shape
    D = input_shape[-1]
    flat = inputs.reshape(-1, D)
    n_tok = flat.shape[0]
    K = W.shape[0]
    bm = _BLOCK_M
    grid = (n_tok // bm,)

    out_shape = [
        jax.ShapeDtypeStruct((n_tok, D), jnp.float32),
        jax.ShapeDtypeStruct((n_tok, K), jnp.float32),
        jax.ShapeDtypeStruct((n_tok, 1), jnp.int32),
        jax.ShapeDtypeStruct((n_tok, K), jnp.float32),
        jax.ShapeDtypeStruct((1, 1), jnp.float32),
        jax.ShapeDtypeStruct((1, 1), jnp.float32),
    ]
    in_specs = [
        pl.BlockSpec((bm, D), lambda i: (i, 0)),
        pl.BlockSpec((K, D), lambda i: (0, 0)),
    ]
    out_specs = [
        pl.BlockSpec((bm, D), lambda i: (i, 0)),
        pl.BlockSpec((bm, K), lambda i: (i, 0)),
        pl.BlockSpec((bm, 1), lambda i: (i, 0)),
        pl.BlockSpec((bm, K), lambda i: (i, 0)),
        pl.BlockSpec((1, 1), lambda i: (0, 0)),
        pl.BlockSpec((1, 1), lambda i: (0, 0)),
    ]
    q_st, enc, idx, dist, loss, ppl = pl.pallas_call(
        _vq_kernel,
        grid=grid,
        in_specs=in_specs,
        out_specs=out_specs,
        out_shape=out_shape,
        scratch_shapes=[
            pltpu.VMEM((1, K), jnp.float32),
            pltpu.VMEM((1, 1), jnp.float32),
        ],
    )(flat, W)

    return (q_st.reshape(input_shape), loss.reshape(()), ppl.reshape(()),
            enc, idx, dist)


# speedup vs baseline: 4.5885x; 1.1244x over previous
import jax
import jax.numpy as jnp
from jax.experimental import pallas as pl
from jax.experimental.pallas import tpu as pltpu

_NUM_EMBEDDINGS = 1024
_EMBEDDING_DIM = 256
_COMMITMENT_COST = 0.25
_BLOCK_M = 1024


def _vq_kernel(x_ref, w_ref, q_ref, enc_ref, idx_ref, dist_ref,
               loss_ref, ppl_ref, cnt_acc, ls_acc):
    i = pl.program_id(0)
    nblk = pl.num_programs(0)
    x = x_ref[...]
    w = w_ref[...]

    x2 = jnp.sum(x * x, axis=1, keepdims=True)
    w2 = jnp.sum(w * w, axis=1)
    xwt = jax.lax.dot_general(
        x, w, (((1,), (1,)), ((), ())),
        preferred_element_type=jnp.float32)
    dist = x2 + w2[None, :] - 2.0 * xwt
    dist_ref[...] = dist

    k_iota = jax.lax.broadcasted_iota(jnp.int32, dist.shape, 1)
    minv = jnp.min(dist, axis=1, keepdims=True)
    idx = jnp.min(jnp.where(dist == minv, k_iota, dist.shape[1]),
                  axis=1).astype(jnp.int32)
    idx_ref[...] = idx[:, None]

    onehot = (k_iota == idx[:, None]).astype(jnp.float32)
    enc_ref[...] = onehot

    q = jax.lax.dot_general(
        onehot, w, (((1,), (0,)), ((), ())),
        preferred_element_type=jnp.float32)
    q_ref[...] = x + (q - x)

    @pl.when(i == 0)
    def _():
        cnt_acc[...] = jnp.zeros_like(cnt_acc)
        ls_acc[...] = jnp.zeros_like(ls_acc)

    diff = q - x
    cnt_acc[...] += jnp.sum(onehot, axis=0, keepdims=True)
    ls_acc[...] += jnp.sum(diff * diff)[None, None]

    @pl.when(i == nblk - 1)
    def _():
        n_tok = nblk * x.shape[0]
        total = n_tok * x.shape[1]
        loss_ref[...] = _COMMITMENT_COST / total * ls_acc[...]
        avg = cnt_acc[...] / n_tok
        ent = jnp.sum(avg * jnp.log(avg + 1e-10), axis=1, keepdims=True)
        ppl_ref[...] = jnp.exp(-ent)


def kernel(inputs, W):
    input_shape = inputs.shape
    D = input_shape[-1]
    flat = inputs.reshape(-1, D)
    n_tok = flat.shape[0]
    K = W.shape[0]
    bm = _BLOCK_M
    grid = (n_tok // bm,)

    out_shape = [
        jax.ShapeDtypeStruct((n_tok, D), jnp.float32),
        jax.ShapeDtypeStruct((n_tok, K), jnp.float32),
        jax.ShapeDtypeStruct((n_tok, 1), jnp.int32),
        jax.ShapeDtypeStruct((n_tok, K), jnp.float32),
        jax.ShapeDtypeStruct((1, 1), jnp.float32),
        jax.ShapeDtypeStruct((1, 1), jnp.float32),
    ]
    in_specs = [
        pl.BlockSpec((bm, D), lambda i: (i, 0)),
        pl.BlockSpec((K, D), lambda i: (0, 0)),
    ]
    out_specs = [
        pl.BlockSpec((bm, D), lambda i: (i, 0)),
        pl.BlockSpec((bm, K), lambda i: (i, 0)),
        pl.BlockSpec((bm, 1), lambda i: (i, 0)),
        pl.BlockSpec((bm, K), lambda i: (i, 0)),
        pl.BlockSpec((1, 1), lambda i: (0, 0)),
        pl.BlockSpec((1, 1), lambda i: (0, 0)),
    ]
    q_st, enc, idx, dist, loss, ppl = pl.pallas_call(
        _vq_kernel,
        grid=grid,
        in_specs=in_specs,
        out_specs=out_specs,
        out_shape=out_shape,
        scratch_shapes=[
            pltpu.VMEM((1, K), jnp.float32),
            pltpu.VMEM((1, 1), jnp.float32),
        ],
    )(flat, W)

    return (q_st.reshape(input_shape), loss.reshape(()), ppl.reshape(()),
            enc, idx, dist)


# speedup vs baseline: 4.7267x; 1.0301x over previous
import jax
import jax.numpy as jnp
from jax.experimental import pallas as pl
from jax.experimental.pallas import tpu as pltpu

_NUM_EMBEDDINGS = 1024
_EMBEDDING_DIM = 256
_COMMITMENT_COST = 0.25
_BLOCK_M = 2048


def _vq_kernel(x_ref, w_ref, q_ref, enc_ref, idx_ref, dist_ref,
               loss_ref, ppl_ref, cnt_acc, ls_acc):
    i = pl.program_id(0)
    nblk = pl.num_programs(0)
    x = x_ref[...]
    w = w_ref[...]

    x2 = jnp.sum(x * x, axis=1, keepdims=True)
    w2 = jnp.sum(w * w, axis=1)
    xwt = jax.lax.dot_general(
        x, w, (((1,), (1,)), ((), ())),
        preferred_element_type=jnp.float32)
    dist = x2 + w2[None, :] - 2.0 * xwt
    dist_ref[...] = dist

    k_iota = jax.lax.broadcasted_iota(jnp.int32, dist.shape, 1)
    minv = jnp.min(dist, axis=1, keepdims=True)
    idx = jnp.min(jnp.where(dist == minv, k_iota, dist.shape[1]),
                  axis=1).astype(jnp.int32)
    idx_ref[...] = idx[:, None]

    onehot = (k_iota == idx[:, None]).astype(jnp.float32)
    enc_ref[...] = onehot

    q = jax.lax.dot_general(
        onehot, w, (((1,), (0,)), ((), ())),
        preferred_element_type=jnp.float32)
    q_ref[...] = x + (q - x)

    @pl.when(i == 0)
    def _():
        cnt_acc[...] = jnp.zeros_like(cnt_acc)
        ls_acc[...] = jnp.zeros_like(ls_acc)

    diff = q - x
    cnt_acc[...] += jnp.sum(onehot, axis=0, keepdims=True)
    ls_acc[...] += jnp.sum(diff * diff)[None, None]

    @pl.when(i == nblk - 1)
    def _():
        n_tok = nblk * x.shape[0]
        total = n_tok * x.shape[1]
        loss_ref[...] = _COMMITMENT_COST / total * ls_acc[...]
        avg = cnt_acc[...] / n_tok
        ent = jnp.sum(avg * jnp.log(avg + 1e-10), axis=1, keepdims=True)
        ppl_ref[...] = jnp.exp(-ent)


def kernel(inputs, W):
    input_shape = inputs.shape
    D = input_shape[-1]
    flat = inputs.reshape(-1, D)
    n_tok = flat.shape[0]
    K = W.shape[0]
    bm = _BLOCK_M
    grid = (n_tok // bm,)

    out_shape = [
        jax.ShapeDtypeStruct((n_tok, D), jnp.float32),
        jax.ShapeDtypeStruct((n_tok, K), jnp.float32),
        jax.ShapeDtypeStruct((n_tok, 1), jnp.int32),
        jax.ShapeDtypeStruct((n_tok, K), jnp.float32),
        jax.ShapeDtypeStruct((1, 1), jnp.float32),
        jax.ShapeDtypeStruct((1, 1), jnp.float32),
    ]
    in_specs = [
        pl.BlockSpec((bm, D), lambda i: (i, 0)),
        pl.BlockSpec((K, D), lambda i: (0, 0)),
    ]
    out_specs = [
        pl.BlockSpec((bm, D), lambda i: (i, 0)),
        pl.BlockSpec((bm, K), lambda i: (i, 0)),
        pl.BlockSpec((bm, 1), lambda i: (i, 0)),
        pl.BlockSpec((bm, K), lambda i: (i, 0)),
        pl.BlockSpec((1, 1), lambda i: (0, 0)),
        pl.BlockSpec((1, 1), lambda i: (0, 0)),
    ]
    q_st, enc, idx, dist, loss, ppl = pl.pallas_call(
        _vq_kernel,
        grid=grid,
        in_specs=in_specs,
        out_specs=out_specs,
        out_shape=out_shape,
        scratch_shapes=[
            pltpu.VMEM((1, K), jnp.float32),
            pltpu.VMEM((1, 1), jnp.float32),
        ],
    )(flat, W)

    return (q_st.reshape(input_shape), loss.reshape(()), ppl.reshape(()),
            enc, idx, dist)


# speedup vs baseline: 4.9634x; 1.0501x over previous
import jax
import jax.numpy as jnp
from jax.experimental import pallas as pl
from jax.experimental.pallas import tpu as pltpu

_NUM_EMBEDDINGS = 1024
_EMBEDDING_DIM = 256
_COMMITMENT_COST = 0.25
_BLOCK_M = 2048


def _vq_kernel(x_ref, w_ref, q_ref, enc_ref, idx_ref, dist_ref,
               loss_ref, ppl_ref, w2_acc, cnt_acc, ls_acc):
    i = pl.program_id(0)
    nblk = pl.num_programs(0)
    x = x_ref[...]
    w = w_ref[...]

    @pl.when(i == 0)
    def _():
        w2_acc[...] = jnp.sum(w * w, axis=1)[None, :]
        cnt_acc[...] = jnp.zeros_like(cnt_acc)
        ls_acc[...] = jnp.zeros_like(ls_acc)

    x2 = jnp.sum(x * x, axis=1, keepdims=True)
    xs = x * (-2.0)
    neg2xwt = jax.lax.dot_general(
        xs, w, (((1,), (1,)), ((), ())),
        preferred_element_type=jnp.float32)
    dist = (x2 + w2_acc[...]) + neg2xwt
    dist_ref[...] = dist

    k_iota = jax.lax.broadcasted_iota(jnp.int32, dist.shape, 1)
    minv = jnp.min(dist, axis=1, keepdims=True)
    idx = jnp.min(jnp.where(dist == minv, k_iota, dist.shape[1]),
                  axis=1).astype(jnp.int32)
    idx_ref[...] = idx[:, None]

    onehot = (k_iota == idx[:, None]).astype(jnp.float32)
    enc_ref[...] = onehot

    q = jax.lax.dot_general(
        onehot, w, (((1,), (0,)), ((), ())),
        preferred_element_type=jnp.float32)
    q_ref[...] = x + (q - x)

    cnt_acc[...] += jnp.sum(onehot, axis=0, keepdims=True)
    ls_acc[...] += jnp.sum(minv)[None, None]

    @pl.when(i == nblk - 1)
    def _():
        n_tok = nblk * x.shape[0]
        total = n_tok * x.shape[1]
        loss_ref[...] = _COMMITMENT_COST / total * ls_acc[...]
        avg = cnt_acc[...] / n_tok
        ent = jnp.sum(avg * jnp.log(avg + 1e-10), axis=1, keepdims=True)
        ppl_ref[...] = jnp.exp(-ent)


def kernel(inputs, W):
    input_shape = inputs.shape
    D = input_shape[-1]
    flat = inputs.reshape(-1, D)
    n_tok = flat.shape[0]
    K = W.shape[0]
    bm = _BLOCK_M
    grid = (n_tok // bm,)

    out_shape = [
        jax.ShapeDtypeStruct((n_tok, D), jnp.float32),
        jax.ShapeDtypeStruct((n_tok, K), jnp.float32),
        jax.ShapeDtypeStruct((n_tok, 1), jnp.int32),
        jax.ShapeDtypeStruct((n_tok, K), jnp.float32),
        jax.ShapeDtypeStruct((1, 1), jnp.float32),
        jax.ShapeDtypeStruct((1, 1), jnp.float32),
    ]
    in_specs = [
        pl.BlockSpec((bm, D), lambda i: (i, 0)),
        pl.BlockSpec((K, D), lambda i: (0, 0)),
    ]
    out_specs = [
        pl.BlockSpec((bm, D), lambda i: (i, 0)),
        pl.BlockSpec((bm, K), lambda i: (i, 0)),
        pl.BlockSpec((bm, 1), lambda i: (i, 0)),
        pl.BlockSpec((bm, K), lambda i: (i, 0)),
        pl.BlockSpec((1, 1), lambda i: (0, 0)),
        pl.BlockSpec((1, 1), lambda i: (0, 0)),
    ]
    q_st, enc, idx, dist, loss, ppl = pl.pallas_call(
        _vq_kernel,
        grid=grid,
        in_specs=in_specs,
        out_specs=out_specs,
        out_shape=out_shape,
        scratch_shapes=[
            pltpu.VMEM((1, K), jnp.float32),
            pltpu.VMEM((1, K), jnp.float32),
            pltpu.VMEM((1, 1), jnp.float32),
        ],
    )(flat, W)

    return (q_st.reshape(input_shape), loss.reshape(()), ppl.reshape(()),
            enc, idx, dist)


# speedup vs baseline: 5.0116x; 1.0097x over previous
import jax
import jax.numpy as jnp
from jax.experimental import pallas as pl
from jax.experimental.pallas import tpu as pltpu

_NUM_EMBEDDINGS = 1024
_EMBEDDING_DIM = 256
_COMMITMENT_COST = 0.25
_BLOCK_M = 2048


def _vq_kernel(x_ref, w_ref, q_ref, enc_ref, idx_ref, dist_ref,
               loss_ref, ppl_ref, w2_acc, cnt_acc, ls_acc):
    i = pl.program_id(0)
    nblk = pl.num_programs(0)
    x3 = x_ref[...]
    x = x3.reshape(-1, x3.shape[-1])
    w = w_ref[...]

    @pl.when(i == 0)
    def _():
        w2_acc[...] = jnp.sum(w * w, axis=1)[None, :]
        cnt_acc[...] = jnp.zeros_like(cnt_acc)
        ls_acc[...] = jnp.zeros_like(ls_acc)

    x2 = jnp.sum(x * x, axis=1, keepdims=True)
    xs = x * (-2.0)
    neg2xwt = jax.lax.dot_general(
        xs, w, (((1,), (1,)), ((), ())),
        preferred_element_type=jnp.float32)
    dist = (x2 + w2_acc[...]) + neg2xwt
    dist_ref[...] = dist

    k_iota = jax.lax.broadcasted_iota(jnp.int32, dist.shape, 1)
    minv = jnp.min(dist, axis=1, keepdims=True)
    idx = jnp.min(jnp.where(dist == minv, k_iota, dist.shape[1]),
                  axis=1).astype(jnp.int32)
    idx_ref[...] = idx[:, None]

    onehot = (k_iota == idx[:, None]).astype(jnp.float32)
    enc_ref[...] = onehot

    q = jax.lax.dot_general(
        onehot, w, (((1,), (0,)), ((), ())),
        preferred_element_type=jnp.float32)
    q_ref[...] = (x + (q - x)).reshape(x3.shape)

    cnt_acc[...] += jnp.sum(onehot, axis=0, keepdims=True)
    ls_acc[...] += jnp.sum(minv)[None, None]

    @pl.when(i == nblk - 1)
    def _():
        n_tok = nblk * x.shape[0]
        total = n_tok * x.shape[1]
        loss_ref[...] = _COMMITMENT_COST / total * ls_acc[...]
        avg = cnt_acc[...] / n_tok
        ent = jnp.sum(avg * jnp.log(avg + 1e-10), axis=1, keepdims=True)
        ppl_ref[...] = jnp.exp(-ent)


def kernel(inputs, W):
    B, S, D = inputs.shape
    n_tok = B * S
    K = W.shape[0]
    bm = _BLOCK_M
    bb = bm // S
    grid = (n_tok // bm,)

    out_shape = [
        jax.ShapeDtypeStruct((B, S, D), jnp.float32),
        jax.ShapeDtypeStruct((n_tok, K), jnp.float32),
        jax.ShapeDtypeStruct((n_tok, 1), jnp.int32),
        jax.ShapeDtypeStruct((n_tok, K), jnp.float32),
        jax.ShapeDtypeStruct((1, 1), jnp.float32),
        jax.ShapeDtypeStruct((1, 1), jnp.float32),
    ]
    in_specs = [
        pl.BlockSpec((bb, S, D), lambda i: (i, 0, 0)),
        pl.BlockSpec((K, D), lambda i: (0, 0)),
    ]
    out_specs = [
        pl.BlockSpec((bb, S, D), lambda i: (i, 0, 0)),
        pl.BlockSpec((bm, K), lambda i: (i, 0)),
        pl.BlockSpec((bm, 1), lambda i: (i, 0)),
        pl.BlockSpec((bm, K), lambda i: (i, 0)),
        pl.BlockSpec((1, 1), lambda i: (0, 0)),
        pl.BlockSpec((1, 1), lambda i: (0, 0)),
    ]
    q_st, enc, idx, dist, loss, ppl = pl.pallas_call(
        _vq_kernel,
        grid=grid,
        in_specs=in_specs,
        out_specs=out_specs,
        out_shape=out_shape,
        scratch_shapes=[
            pltpu.VMEM((1, K), jnp.float32),
            pltpu.VMEM((1, K), jnp.float32),
            pltpu.VMEM((1, 1), jnp.float32),
        ],
    )(inputs, W)

    return (q_st, loss.reshape(()), ppl.reshape(()), enc, idx, dist)


# speedup vs baseline: 5.2913x; 1.0558x over previous
import jax
import jax.numpy as jnp
from jax.experimental import pallas as pl
from jax.experimental.pallas import tpu as pltpu

_NUM_EMBEDDINGS = 1024
_EMBEDDING_DIM = 256
_COMMITMENT_COST = 0.25
_BLOCK_M = 2048


def _vq_kernel(x_ref, w_ref, q_ref, enc_ref, idx_ref, dist_ref,
               loss_ref, ppl_ref, w2_acc, cnt_acc, ls_acc):
    i = pl.program_id(0)
    nblk = pl.num_programs(0)
    x3 = x_ref[...]
    x = x3.reshape(-1, x3.shape[-1])
    w = w_ref[...]

    @pl.when(i == 0)
    def _():
        w2_acc[...] = jnp.sum(w * w, axis=1)[None, :]
        cnt_acc[...] = jnp.zeros_like(cnt_acc)
        ls_acc[...] = jnp.zeros_like(ls_acc)

    x2 = jnp.sum(x * x, axis=1, keepdims=True)
    xs = x * (-2.0)
    neg2xwt = jax.lax.dot_general(
        xs, w, (((1,), (1,)), ((), ())),
        preferred_element_type=jnp.float32)
    dist = (x2 + w2_acc[...]) + neg2xwt
    dist_ref[...] = dist

    kf_iota = jax.lax.broadcasted_iota(
        jnp.int32, (1, dist.shape[1]), 1).astype(jnp.float32)
    minv = jnp.min(dist, axis=1, keepdims=True)
    idxf = jnp.min(jnp.where(dist == minv, kf_iota, float(dist.shape[1])),
                   axis=1, keepdims=True)
    idx_ref[...] = idxf.astype(jnp.int32)

    onehot = (kf_iota == idxf).astype(jnp.float32)
    enc_ref[...] = onehot

    q = jax.lax.dot_general(
        onehot, w, (((1,), (0,)), ((), ())),
        preferred_element_type=jnp.float32)
    q_ref[...] = (x + (q - x)).reshape(x3.shape)

    ones_row = jnp.ones((1, onehot.shape[0]), jnp.float32)
    cnt_acc[...] += jax.lax.dot_general(
        ones_row, onehot, (((1,), (0,)), ((), ())),
        preferred_element_type=jnp.float32)
    ls_acc[...] += jnp.sum(minv)[None, None]

    @pl.when(i == nblk - 1)
    def _():
        n_tok = nblk * x.shape[0]
        total = n_tok * x.shape[1]
        loss_ref[...] = _COMMITMENT_COST / total * ls_acc[...]
        avg = cnt_acc[...] / n_tok
        ent = jnp.sum(avg * jnp.log(avg + 1e-10), axis=1, keepdims=True)
        ppl_ref[...] = jnp.exp(-ent)


def kernel(inputs, W):
    B, S, D = inputs.shape
    n_tok = B * S
    K = W.shape[0]
    bm = _BLOCK_M
    bb = bm // S
    grid = (n_tok // bm,)

    out_shape = [
        jax.ShapeDtypeStruct((B, S, D), jnp.float32),
        jax.ShapeDtypeStruct((n_tok, K), jnp.float32),
        jax.ShapeDtypeStruct((n_tok, 1), jnp.int32),
        jax.ShapeDtypeStruct((n_tok, K), jnp.float32),
        jax.ShapeDtypeStruct((1, 1), jnp.float32),
        jax.ShapeDtypeStruct((1, 1), jnp.float32),
    ]
    in_specs = [
        pl.BlockSpec((bb, S, D), lambda i: (i, 0, 0)),
        pl.BlockSpec((K, D), lambda i: (0, 0)),
    ]
    out_specs = [
        pl.BlockSpec((bb, S, D), lambda i: (i, 0, 0)),
        pl.BlockSpec((bm, K), lambda i: (i, 0)),
        pl.BlockSpec((bm, 1), lambda i: (i, 0)),
        pl.BlockSpec((bm, K), lambda i: (i, 0)),
        pl.BlockSpec((1, 1), lambda i: (0, 0)),
        pl.BlockSpec((1, 1), lambda i: (0, 0)),
    ]
    q_st, enc, idx, dist, loss, ppl = pl.pallas_call(
        _vq_kernel,
        grid=grid,
        in_specs=in_specs,
        out_specs=out_specs,
        out_shape=out_shape,
        scratch_shapes=[
            pltpu.VMEM((1, K), jnp.float32),
            pltpu.VMEM((1, K), jnp.float32),
            pltpu.VMEM((1, 1), jnp.float32),
        ],
    )(inputs, W)

    return (q_st, loss.reshape(()), ppl.reshape(()), enc, idx, dist)
